# Initial kernel scaffold; baseline (speedup 1.0000x reference)
import functools

import jax
import jax.numpy as jnp
from jax import lax
from jax.experimental import pallas as pl
from jax.experimental.pallas import tpu as pltpu
from jax.experimental.pallas import tpu_sc as plsc

N_USERS = 10000
N_ITEMS = 40000
N_NODES = N_USERS + N_ITEMS
EMB = 64
HALF = EMB // 2
N_EDGES = 800000
NC = 2
NS = 16
LANES = 16

C = 100
K = 100
EPT = N_EDGES // NS
CPT = EPT // C
NSUP = CPT // K
RPT = N_NODES // NS
RB = 625
NRB = RPT // RB


def _layer_body(col_hbm, row_hbm, val_hbm, x_hbm, out_hbm,
                acc, colbuf, rowbuf, valbuf, gb0, gb1, zbuf, sem0, sem1):
    cid = lax.axis_index("c")
    sid = lax.axis_index("s")

    def zrow(i, _):
        zbuf[i, pl.ds(0, LANES)] = jnp.zeros((LANES,), jnp.float32)
        zbuf[i, pl.ds(LANES, LANES)] = jnp.zeros((LANES,), jnp.float32)
        return 0
    lax.fori_loop(0, RB, zrow, 0)
    r0 = sid * RPT
    for z in range(NRB):
        pltpu.sync_copy(zbuf, acc.at[pl.ds(r0 + z * RB, RB)])
    plsc.subcore_barrier()

    tile_chunk0 = sid * CPT

    def issue(j, gb, sem):
        pltpu.async_copy(x_hbm.at[cid, colbuf.at[j]], gb, sem)

    def scale_scatter(j, gb):
        def sbody(e, _):
            v = valbuf[j, e]
            gb[e, pl.ds(0, LANES)] = gb[e, pl.ds(0, LANES)] * v
            gb[e, pl.ds(LANES, LANES)] = gb[e, pl.ds(LANES, LANES)] * v
            return 0
        lax.fori_loop(0, C, sbody, 0)
        pltpu.sync_copy(gb, acc.at[rowbuf.at[j]], add=True)

    for s in range(NSUP):
        sup0 = tile_chunk0 + s * K
        pltpu.sync_copy(col_hbm.at[pl.ds(sup0, K)], colbuf)
        pltpu.sync_copy(row_hbm.at[pl.ds(sup0, K)], rowbuf)
        pltpu.sync_copy(val_hbm.at[pl.ds(sup0, K)], valbuf)

        issue(0, gb0, sem0)

        def pair(j2, _):
            j = 2 * j2
            issue(j + 1, gb1, sem1)
            pltpu.make_async_copy(x_hbm.at[cid, colbuf.at[j]], gb0, sem0).wait()
            scale_scatter(j, gb0)

            @pl.when(j2 < K // 2 - 1)
            def _():
                issue(j + 2, gb0, sem0)
            pltpu.make_async_copy(x_hbm.at[cid, colbuf.at[j + 1]], gb1, sem1).wait()
            scale_scatter(j + 1, gb1)
            return 0
        lax.fori_loop(0, K // 2, pair, 0)

    plsc.subcore_barrier()
    for z in range(NRB):
        pltpu.sync_copy(acc.at[pl.ds(r0 + z * RB, RB)], zbuf)
        pltpu.sync_copy(zbuf, out_hbm.at[cid, pl.ds(r0 + z * RB, RB)])


_layer = functools.partial(
    pl.kernel,
    out_type=jax.ShapeDtypeStruct((NC, N_NODES, HALF), jnp.float32),
    mesh=plsc.VectorSubcoreMesh(
        core_axis_name="c", subcore_axis_name="s",
        num_cores=NC, num_subcores=NS),
    scratch_types=[
        pltpu.VMEM_SHARED((N_NODES, HALF), jnp.float32),
        pltpu.VMEM((K, C), jnp.int32),
        pltpu.VMEM((K, C), jnp.int32),
        pltpu.VMEM((K, C), jnp.float32),
        pltpu.VMEM((C, HALF), jnp.float32),
        pltpu.VMEM((C, HALF), jnp.float32),
        pltpu.VMEM((RB, HALF), jnp.float32),
        pltpu.SemaphoreType.DMA,
        pltpu.SemaphoreType.DMA,
    ],
)(_layer_body)


def _mean_body(a, b, c, d, o):
    o[...] = (a[...] + b[...] + c[...] + d[...]) * 0.25


_FLAT = NC * N_NODES * HALF
_MROWS = _FLAT // 128
_MBLK = 1000

_mean4 = pl.pallas_call(
    _mean_body,
    grid=(_MROWS // _MBLK,),
    in_specs=[pl.BlockSpec((_MBLK, 128), lambda i: (i, 0))] * 4,
    out_specs=pl.BlockSpec((_MBLK, 128), lambda i: (i, 0)),
    out_shape=jax.ShapeDtypeStruct((_MROWS, 128), jnp.float32),
)


def kernel(adj_indices, adj_values, user_emb, item_emb):
    all_emb = jnp.concatenate([user_emb, item_emb], axis=0)
    x0 = jnp.stack([all_emb[:, :HALF], all_emb[:, HALF:]])
    row = adj_indices[0].astype(jnp.int32).reshape(N_EDGES // C, C)
    col = adj_indices[1].astype(jnp.int32).reshape(N_EDGES // C, C)
    val = adj_values.reshape(N_EDGES // C, C)

    x1 = _layer(col, row, val, x0)
    x2 = _layer(col, row, val, x1)
    x3 = _layer(col, row, val, x2)

    m = _mean4(x0.reshape(_MROWS, 128), x1.reshape(_MROWS, 128),
               x2.reshape(_MROWS, 128), x3.reshape(_MROWS, 128))
    m = m.reshape(NC, N_NODES, HALF)
    full = jnp.concatenate([m[0], m[1]], axis=1)
    return (full[:N_USERS], full[N_USERS:])


# speedup vs baseline: 7.2446x; 7.2446x over previous
import functools

import jax
import jax.numpy as jnp
from jax import lax
from jax.experimental import pallas as pl
from jax.experimental.pallas import tpu as pltpu
from jax.experimental.pallas import tpu_sc as plsc

N_USERS = 10000
N_ITEMS = 40000
N_NODES = N_USERS + N_ITEMS
EMB = 64
HALF = EMB // 2
N_EDGES = 800000
NC = 2
NS = 16
LANES = 16

C = 128
K = 40
E_PAD = 819200
CPT = E_PAD // NS // C
NSUP = CPT // K
RB = 200
NRBLK = N_NODES // RB


def _layer_body(col_hbm, row_hbm, val_hbm, x_hbm, out_hbm,
                acc, colbuf, rowbuf, valbuf, gb0, gb1, zbuf, sem0, sem1):
    cid = lax.axis_index("c")
    sid = lax.axis_index("s")

    def zrow(i, _):
        zbuf[i, pl.ds(0, LANES)] = jnp.zeros((LANES,), jnp.float32)
        zbuf[i, pl.ds(LANES, LANES)] = jnp.zeros((LANES,), jnp.float32)
        return 0
    lax.fori_loop(0, RB, zrow, 0)

    def blk_off(i):
        return pl.multiple_of((i * NS + sid) * RB, RB)

    for i in range(-(-NRBLK // NS)):
        @pl.when(i * NS + sid < NRBLK)
        def _():
            pltpu.sync_copy(zbuf, acc.at[pl.ds(blk_off(i), RB)])
    plsc.subcore_barrier()

    tile_chunk0 = sid * CPT

    def issue(j, gb, sem):
        pltpu.async_copy(x_hbm.at[cid].at[colbuf.at[j]], gb, sem)

    def scale_scatter(j, gb):
        def group(g, _):
            vvec = valbuf[j, pl.ds(LANES * g, LANES)]
            for ei in range(LANES):
                e = LANES * g + ei
                v = vvec[ei]
                gb[e, pl.ds(0, LANES)] = gb[e, pl.ds(0, LANES)] * v
                gb[e, pl.ds(LANES, LANES)] = gb[e, pl.ds(LANES, LANES)] * v
            return 0
        lax.fori_loop(0, C // LANES, group, 0)
        pltpu.sync_copy(gb, acc.at[rowbuf.at[j]], add=True)

    for s in range(NSUP):
        sup0 = pl.multiple_of(tile_chunk0 + s * K, K)
        pltpu.sync_copy(col_hbm.at[pl.ds(sup0, K)], colbuf)
        pltpu.sync_copy(row_hbm.at[pl.ds(sup0, K)], rowbuf)
        pltpu.sync_copy(val_hbm.at[pl.ds(sup0, K)], valbuf)

        issue(0, gb0, sem0)

        def pair(j2, _):
            j = 2 * j2
            issue(j + 1, gb1, sem1)
            pltpu.make_async_copy(x_hbm.at[cid].at[colbuf.at[j]], gb0, sem0).wait()
            scale_scatter(j, gb0)

            @pl.when(j2 < K // 2 - 1)
            def _():
                issue(j + 2, gb0, sem0)
            pltpu.make_async_copy(x_hbm.at[cid].at[colbuf.at[j + 1]], gb1, sem1).wait()
            scale_scatter(j + 1, gb1)
            return 0
        lax.fori_loop(0, K // 2, pair, 0)

    plsc.subcore_barrier()
    for i in range(-(-NRBLK // NS)):
        @pl.when(i * NS + sid < NRBLK)
        def _():
            off = blk_off(i)
            pltpu.sync_copy(acc.at[pl.ds(off, RB)], zbuf)
            pltpu.sync_copy(zbuf, out_hbm.at[cid, pl.ds(off, RB)])


_layer = functools.partial(
    pl.kernel,
    out_type=jax.ShapeDtypeStruct((NC, N_NODES, HALF), jnp.float32),
    mesh=plsc.VectorSubcoreMesh(
        core_axis_name="c", subcore_axis_name="s",
        num_cores=NC, num_subcores=NS),
    scratch_types=[
        pltpu.VMEM_SHARED((N_NODES, HALF), jnp.float32),
        pltpu.VMEM((K, C), jnp.int32),
        pltpu.VMEM((K, C), jnp.int32),
        pltpu.VMEM((K, C), jnp.float32),
        pltpu.VMEM((C, HALF), jnp.float32),
        pltpu.VMEM((C, HALF), jnp.float32),
        pltpu.VMEM((RB, HALF), jnp.float32),
        pltpu.SemaphoreType.DMA,
        pltpu.SemaphoreType.DMA,
    ],
    compiler_params=pltpu.CompilerParams(use_tc_tiling_on_sc=False),
)(_layer_body)


def _mean_body(a, b, c, d, o):
    o[...] = (a[...] + b[...] + c[...] + d[...]) * 0.25


_FLAT = NC * N_NODES * HALF
_MROWS = _FLAT // 128
_MBLK = 1000

_mean4 = pl.pallas_call(
    _mean_body,
    grid=(_MROWS // _MBLK,),
    in_specs=[pl.BlockSpec((_MBLK, 128), lambda i: (i, 0))] * 4,
    out_specs=pl.BlockSpec((_MBLK, 128), lambda i: (i, 0)),
    out_shape=jax.ShapeDtypeStruct((_MROWS, 128), jnp.float32),
)


def kernel(adj_indices, adj_values, user_emb, item_emb):
    all_emb = jnp.concatenate([user_emb, item_emb], axis=0)
    x0 = jnp.stack([all_emb[:, :HALF], all_emb[:, HALF:]])
    npad = E_PAD - N_EDGES
    idx = adj_indices.astype(jnp.int32)
    row = jnp.concatenate([idx[0], jnp.zeros((npad,), jnp.int32)])
    col = jnp.concatenate([idx[1], jnp.zeros((npad,), jnp.int32)])
    val = jnp.concatenate([adj_values, jnp.zeros((npad,), jnp.float32)])
    row = row.reshape(E_PAD // C, C)
    col = col.reshape(E_PAD // C, C)
    val = val.reshape(E_PAD // C, C)

    x1 = _layer(col, row, val, x0)
    x2 = _layer(col, row, val, x1)
    x3 = _layer(col, row, val, x2)

    m = _mean4(x0.reshape(_MROWS, 128), x1.reshape(_MROWS, 128),
               x2.reshape(_MROWS, 128), x3.reshape(_MROWS, 128))
    m = m.reshape(NC, N_NODES, HALF)
    full = jnp.concatenate([m[0], m[1]], axis=1)
    return (full[:N_USERS], full[N_USERS:])
